# Initial kernel scaffold; baseline (speedup 1.0000x reference)
import functools

import numpy as np
import jax
import jax.numpy as jnp
from jax import lax
from jax.experimental import pallas as pl
from jax.experimental.pallas import tpu as pltpu
from jax.experimental.pallas import tpu_sc as plsc

N = 100000
E = 1600000
G = 64
EPS = 1e-5
F32 = jnp.float32

CH = 128
NCHUNK = E // CH
GRP = 8
BN = 2000
NB = N // BN
ROWS_PER_TILE = N // 16
WB = 125

_MESH = dict(core_axis_name="c", subcore_axis_name="s", num_cores=2,
             num_subcores=16)


def _fill_rows(ref, nrows, value):
    row = jnp.full((16,), value, F32)
    for i in range(nrows):
        ref[i, :] = row


def _zero_acc_rows(acc, zbuf, r0):
    def zc(j, carry):
        pltpu.sync_copy(zbuf, acc.at[pl.ds(r0 + j * WB, WB), :])
        return carry
    lax.fori_loop(0, ROWS_PER_TILE // WB, zc, 0)


def _writeback_rows(acc, wbuf, out, r0):
    def wb(j, carry):
        rows = pl.ds(r0 + j * WB, WB)
        pltpu.sync_copy(acc.at[rows, :], wbuf)
        pltpu.sync_copy(wbuf, out.at[rows, :])
        return carry
    lax.fori_loop(0, ROWS_PER_TILE // WB, wb, 0)


_DEG_HALF = NCHUNK // 2
_DEG_PER_TILE = _DEG_HALF // 16
_DEG_REM = _DEG_HALF - 16 * _DEG_PER_TILE
_DEG_FULL_GROUPS = _DEG_PER_TILE // GRP
_DEG_TAIL = _DEG_PER_TILE + 1 - _DEG_FULL_GROUPS * GRP


@functools.partial(
    pl.kernel,
    out_type=[jax.ShapeDtypeStruct((N, 16), F32)] * 2,
    mesh=plsc.VectorSubcoreMesh(**_MESH),
    scratch_types=[
        pltpu.VMEM_SHARED((N, 16), F32),
        pltpu.VMEM((GRP, CH), jnp.int32),
        pltpu.VMEM((CH, 16), F32),
        pltpu.VMEM((WB, 16), F32),
        pltpu.VMEM((WB, 16), F32),
    ],
)
def _sc_deg(dst2, deg_a, deg_b, acc, dbig, ones, zbuf, wbuf):
    c = lax.axis_index("c")
    s = lax.axis_index("s")
    _fill_rows(ones, CH, 1.0)
    _fill_rows(zbuf, WB, 0.0)
    r0 = s * ROWS_PER_TILE
    cnt = _DEG_PER_TILE + jnp.where(s < _DEG_REM, 1, 0)
    start = c * _DEG_HALF + s * _DEG_PER_TILE + jnp.minimum(s, _DEG_REM)

    _zero_acc_rows(acc, zbuf, r0)
    plsc.subcore_barrier()

    def group(g, carry):
        g0 = start + g * GRP
        pltpu.sync_copy(dst2.at[pl.ds(g0, GRP), :], dbig)
        for b in range(GRP):
            pltpu.sync_copy(ones, acc.at[dbig.at[b]], add=True)
        return carry

    lax.fori_loop(0, _DEG_FULL_GROUPS, group, 0)

    t0 = _DEG_FULL_GROUPS * GRP
    pltpu.sync_copy(dst2.at[pl.ds(start + t0, GRP), :], dbig)
    for t in range(_DEG_TAIL):
        @pl.when(t0 + t < cnt)
        def _(t=t):
            pltpu.sync_copy(ones, acc.at[dbig.at[t]], add=True)

    plsc.subcore_barrier()

    @pl.when(c == 0)
    def _():
        _writeback_rows(acc, wbuf, deg_a, r0)

    @pl.when(c == 1)
    def _():
        _writeback_rows(acc, wbuf, deg_b, r0)


_PER_TILE = NCHUNK // 16
_REM = NCHUNK - 16 * _PER_TILE
_FULL_GROUPS = _PER_TILE // GRP
_TAIL = _PER_TILE + 1 - _FULL_GROUPS * GRP


def _make_sc_scatter(fbn):
    bps = fbn // 2

    scratch = [
        pltpu.VMEM_SHARED((N, 16), F32),
        pltpu.VMEM((GRP, CH), jnp.int32),
        pltpu.VMEM((GRP, CH), jnp.int32),
        pltpu.VMEM((CH, 16), F32),
        pltpu.VMEM((CH, 16), F32),
        pltpu.VMEM((WB, 16), F32),
        pltpu.VMEM((WB, 16), F32),
        pltpu.SemaphoreType.DMA,
        pltpu.SemaphoreType.DMA,
    ]

    @functools.partial(
        pl.kernel,
        out_type=[jax.ShapeDtypeStruct((N, 16), F32)] * fbn,
        mesh=plsc.VectorSubcoreMesh(**_MESH),
        scratch_types=scratch,
    )
    def k(src2, dst2, *rest):
        tabs = rest[:fbn]
        outs = rest[fbn:2 * fbn]
        acc, sbig, dbig, u0, u1, zbuf, wbuf, g0s, g1s = rest[2 * fbn:]
        ubufs = (u0, u1)
        gsems = (g0s, g1s)

        c = lax.axis_index("c")
        s = lax.axis_index("s")
        _fill_rows(zbuf, WB, 0.0)
        r0 = s * ROWS_PER_TILE
        cnt = _PER_TILE + jnp.where(s < _REM, 1, 0)
        start = s * _PER_TILE + jnp.minimum(s, _REM)

        for fb in range(fbn):
            @pl.when(c == fb // bps)
            def _(fb=fb):
                tab = tabs[fb]
                out = outs[fb]
                _zero_acc_rows(acc, zbuf, r0)
                plsc.subcore_barrier()

                def group(g, carry):
                    g0 = start + g * GRP
                    pltpu.sync_copy(src2.at[pl.ds(g0, GRP), :], sbig)
                    pltpu.sync_copy(dst2.at[pl.ds(g0, GRP), :], dbig)
                    d_prev = pltpu.async_copy(tab.at[sbig.at[0]], ubufs[0],
                                              gsems[0])
                    for b in range(1, GRP):
                        d_cur = pltpu.async_copy(tab.at[sbig.at[b]],
                                                 ubufs[b % 2], gsems[b % 2])
                        d_prev.wait()
                        pltpu.sync_copy(ubufs[(b - 1) % 2],
                                        acc.at[dbig.at[b - 1]], add=True)
                        d_prev = d_cur
                    d_prev.wait()
                    pltpu.sync_copy(ubufs[(GRP - 1) % 2],
                                    acc.at[dbig.at[GRP - 1]], add=True)
                    return carry

                lax.fori_loop(0, _FULL_GROUPS, group, 0)

                t0 = _FULL_GROUPS * GRP
                pltpu.sync_copy(src2.at[pl.ds(start + t0, GRP), :], sbig)
                pltpu.sync_copy(dst2.at[pl.ds(start + t0, GRP), :], dbig)
                for t in range(_TAIL):
                    @pl.when(t0 + t < cnt)
                    def _(t=t):
                        pltpu.async_copy(tab.at[sbig.at[t]], ubufs[0],
                                         gsems[0]).wait()
                        pltpu.sync_copy(ubufs[0], acc.at[dbig.at[t]],
                                        add=True)

                plsc.subcore_barrier()
                _writeback_rows(acc, wbuf, out, r0)

    return k


_sc_scatter4 = _make_sc_scatter(4)
_sc_scatter8 = _make_sc_scatter(8)


def _row_specs(width, n=1):
    return [pl.BlockSpec((BN, width), lambda i: (i, 0)) for _ in range(n)]


def _tc_pre1(deg_a, deg_b, x, w1):
    def body(da, db, xb, w, dinv_o, *houts):
        deg = da[:, 0:1] + db[:, 0:1] + 1.0
        dinv = lax.rsqrt(deg)
        h = jnp.dot(xb[...], w[...], preferred_element_type=F32,
                    precision=lax.Precision.HIGHEST) * dinv
        dinv_o[...] = dinv
        for i, o in enumerate(houts):
            o[...] = h[:, 16 * i:16 * (i + 1)]

    return pl.pallas_call(
        body,
        grid=(NB,),
        in_specs=(_row_specs(16) + _row_specs(16) + _row_specs(22)
                  + [pl.BlockSpec((22, 64), lambda i: (0, 0))]),
        out_specs=_row_specs(1) + _row_specs(16, 4),
        out_shape=([jax.ShapeDtypeStruct((N, 1), F32)]
                   + [jax.ShapeDtypeStruct((N, 16), F32)] * 4),
    )(deg_a, deg_b, x, w1)


def _tc_mid(S, hp, dinv, bvec, gvec, bevec, w, fbo):
    fout = 16 * fbo

    def body(*refs):
        s_refs = refs[0:4]
        h_refs = refs[4:8]
        dinv_r, b_r, g_r, be_r, w_r = refs[8:13]
        outs = refs[13:]
        sc = jnp.concatenate([r[...] for r in s_refs], axis=1)
        hc = jnp.concatenate([r[...] for r in h_refs], axis=1)
        dv = dinv_r[...]
        pre = dv * (sc + hc) + b_r[...]
        y = jnp.maximum(pre * (g_r[...] * (1.0 / np.sqrt(1.0 + EPS)))
                        + be_r[...], 0.0)
        hn = jnp.dot(y, w_r[...], preferred_element_type=F32,
                     precision=lax.Precision.HIGHEST) * dv
        for i, o in enumerate(outs):
            o[...] = hn[:, 16 * i:16 * (i + 1)]

    return pl.pallas_call(
        body,
        grid=(NB,),
        in_specs=(_row_specs(16, 4) + _row_specs(16, 4) + _row_specs(1)
                  + [pl.BlockSpec((1, 64), lambda i: (0, 0))] * 3
                  + [pl.BlockSpec((64, fout), lambda i: (0, 0))]),
        out_specs=_row_specs(16, fbo),
        out_shape=[jax.ShapeDtypeStruct((N, 16), F32)] * fbo,
    )(*S, *hp, dinv, bvec, gvec, bevec, w)


def _tc_final(S, hp, dinv, b3, batch3):
    def body(*refs):
        s_refs = refs[0:8]
        h_refs = refs[8:16]
        dinv_r, b_r, batch_r, out_ref, acc, cntr = refs[16:]
        i = pl.program_id(0)

        @pl.when(i == 0)
        def _():
            acc[...] = jnp.zeros_like(acc)
            cntr[...] = jnp.zeros_like(cntr)

        sc = jnp.concatenate([r[...] for r in s_refs], axis=1)
        hc = jnp.concatenate([r[...] for r in h_refs], axis=1)
        pre = dinv_r[...] * (sc + hc) + b_r[...]
        y = jnp.maximum(pre, 0.0)
        oh = (lax.broadcasted_iota(jnp.int32, (G, BN), 0)
              == batch_r[0]).astype(F32)
        acc[...] += jnp.dot(oh, y, preferred_element_type=F32,
                            precision=lax.Precision.HIGHEST)
        cntr[...] += jnp.broadcast_to(
            jnp.sum(oh, axis=1, keepdims=True), (G, 128))
        out_ref[...] = acc[...] / jnp.maximum(cntr[...], 1.0)

    return pl.pallas_call(
        body,
        grid=(NB,),
        in_specs=(_row_specs(16, 8) + _row_specs(16, 8) + _row_specs(1)
                  + [pl.BlockSpec((1, 128), lambda i: (0, 0)),
                     pl.BlockSpec((1, 1, BN), lambda i: (i, 0, 0))]),
        out_specs=pl.BlockSpec((G, 128), lambda i: (0, 0)),
        out_shape=jax.ShapeDtypeStruct((G, 128), F32),
        scratch_shapes=[pltpu.VMEM((G, 128), F32), pltpu.VMEM((G, 128), F32)],
    )(*S, *hp, dinv, b3, batch3)


def kernel(x, edge_index, batch, W1, b1, g1, be1, W2, b2, g2, be2, W3, b3):
    src2 = jnp.pad(edge_index[0].reshape(NCHUNK, CH), ((0, 4), (0, 0)))
    dst2 = jnp.pad(edge_index[1].reshape(NCHUNK, CH), ((0, 4), (0, 0)))

    deg_a, deg_b = _sc_deg(dst2)
    dinv, *hp1 = _tc_pre1(deg_a, deg_b, x, W1)
    s1 = _sc_scatter4(src2, dst2, *hp1)
    hp2 = _tc_mid(s1, hp1, dinv, b1.reshape(1, 64), g1.reshape(1, 64),
                  be1.reshape(1, 64), W2, 4)
    s2 = _sc_scatter4(src2, dst2, *hp2)
    hp3 = _tc_mid(s2, hp2, dinv, b2.reshape(1, 64), g2.reshape(1, 64),
                  be2.reshape(1, 64), W3, 8)
    s3 = _sc_scatter8(src2, dst2, *hp3)
    return _tc_final(s3, hp3, dinv, b3.reshape(1, 128),
                     batch.reshape(NB, 1, BN))


# speedup vs baseline: 8.2743x; 8.2743x over previous
import functools

import numpy as np
import jax
import jax.numpy as jnp
from jax import lax
from jax.experimental import pallas as pl
from jax.experimental.pallas import tpu as pltpu
from jax.experimental.pallas import tpu_sc as plsc

N = 100000
NP = 100352
E = 1600000
G = 64
EPS = 1e-5
F32 = jnp.float32

CH = 128
NCHUNK = E // CH
GRP = 8
NGF = NCHUNK // GRP
PARTIAL = NCHUNK - NGF * GRP
NCHUNK_PAD = (NGF + 1) * GRP

BN = 2048
NB = NP // BN
RPT = NP // 16
WB = 128
NWB = RPT // WB

_MESH = dict(core_axis_name="c", subcore_axis_name="s", num_cores=2,
             num_subcores=16)


def _fill_rows(ref, nrows, value):
    row = jnp.full((16,), value, F32)
    for i in range(nrows):
        ref[i, :] = row


def _zero_acc_rows(acc, zbuf, r0):
    def zc(j, carry):
        pltpu.sync_copy(zbuf, acc.at[pl.ds(r0 + j * WB, WB), :])
        return carry
    lax.fori_loop(0, NWB, zc, 0)


def _writeback_rows(acc, wbuf, out, r0):
    def wb(j, carry):
        rows = pl.ds(r0 + j * WB, WB)
        pltpu.sync_copy(acc.at[rows, :], wbuf)
        pltpu.sync_copy(wbuf, out.at[rows, :])
        return carry
    lax.fori_loop(0, NWB, wb, 0)


_DEG_HALF_G = NGF // 2
_DEG_PER_TILE_G = _DEG_HALF_G // 16
_DEG_REM_G = _DEG_HALF_G - 16 * _DEG_PER_TILE_G


def _sc_deg_body(dst2, deg_a, deg_b, acc, dbig, ones, zbuf, wbuf):
    c = lax.axis_index("c")
    s = lax.axis_index("s")
    _fill_rows(ones, CH, 1.0)
    _fill_rows(zbuf, WB, 0.0)
    r0 = s * RPT
    cnt_g = _DEG_PER_TILE_G + jnp.where(s < _DEG_REM_G, 1, 0)
    start_g = c * _DEG_HALF_G + s * _DEG_PER_TILE_G + jnp.minimum(s, _DEG_REM_G)

    _zero_acc_rows(acc, zbuf, r0)
    plsc.subcore_barrier()

    def group(g, carry):
        pltpu.sync_copy(dst2.at[pl.ds((start_g + g) * GRP, GRP), :], dbig)
        for b in range(GRP):
            pltpu.sync_copy(ones, acc.at[dbig.at[b]], add=True)
        return carry

    lax.fori_loop(0, cnt_g, group, 0)

    @pl.when(jnp.logical_and(c == 1, s == 15))
    def _():
        pltpu.sync_copy(dst2.at[pl.ds(NGF * GRP, GRP), :], dbig)
        for b in range(PARTIAL):
            pltpu.sync_copy(ones, acc.at[dbig.at[b]], add=True)

    plsc.subcore_barrier()

    @pl.when(c == 0)
    def _():
        _writeback_rows(acc, wbuf, deg_a, r0)

    @pl.when(c == 1)
    def _():
        _writeback_rows(acc, wbuf, deg_b, r0)


@functools.cache
def _make_sc_deg():
    return functools.partial(
        pl.kernel,
        out_type=[jax.ShapeDtypeStruct((NP, 16), F32)] * 2,
        mesh=plsc.VectorSubcoreMesh(**_MESH),
        scratch_types=[
            pltpu.VMEM_SHARED((NP, 16), F32),
            pltpu.VMEM((GRP, CH), jnp.int32),
            pltpu.VMEM((CH, 16), F32),
            pltpu.VMEM((WB, 16), F32),
            pltpu.VMEM((WB, 16), F32),
        ],
        compiler_params=pltpu.CompilerParams(use_tc_tiling_on_sc=False),
    )(_sc_deg_body)


_PER_TILE_G = NGF // 16
_REM_G = NGF - 16 * _PER_TILE_G


@functools.cache
def _make_sc_scatter(fbn):
    bps = fbn // 2

    scratch = [
        pltpu.VMEM_SHARED((NP, 16), F32),
        pltpu.VMEM((GRP, CH), jnp.int32),
        pltpu.VMEM((GRP, CH), jnp.int32),
        pltpu.VMEM((CH, 16), F32),
        pltpu.VMEM((CH, 16), F32),
        pltpu.VMEM((WB, 16), F32),
        pltpu.VMEM((WB, 16), F32),
        pltpu.SemaphoreType.DMA,
        pltpu.SemaphoreType.DMA,
    ]

    @functools.partial(
        pl.kernel,
        out_type=[jax.ShapeDtypeStruct((NP, 16), F32)] * fbn,
        mesh=plsc.VectorSubcoreMesh(**_MESH),
        scratch_types=scratch,
        compiler_params=pltpu.CompilerParams(use_tc_tiling_on_sc=False),
    )
    def k(src2, dst2, *rest):
        tabs = rest[:fbn]
        outs = rest[fbn:2 * fbn]
        acc, sbig, dbig, u0, u1, zbuf, wbuf, g0s, g1s = rest[2 * fbn:]
        ubufs = (u0, u1)
        gsems = (g0s, g1s)

        c = lax.axis_index("c")
        s = lax.axis_index("s")
        _fill_rows(zbuf, WB, 0.0)
        r0 = s * RPT
        cnt_g = _PER_TILE_G + jnp.where(s < _REM_G, 1, 0)
        start_g = s * _PER_TILE_G + jnp.minimum(s, _REM_G)

        for fb in range(fbn):
            @pl.when(c == fb // bps)
            def _(fb=fb):
                tab = tabs[fb]
                out = outs[fb]
                _zero_acc_rows(acc, zbuf, r0)
                plsc.subcore_barrier()

                def group(g, carry):
                    row0 = (start_g + g) * GRP
                    pltpu.sync_copy(src2.at[pl.ds(row0, GRP), :], sbig)
                    pltpu.sync_copy(dst2.at[pl.ds(row0, GRP), :], dbig)
                    d_prev = pltpu.async_copy(tab.at[sbig.at[0]], ubufs[0],
                                              gsems[0])
                    for b in range(1, GRP):
                        d_cur = pltpu.async_copy(tab.at[sbig.at[b]],
                                                 ubufs[b % 2], gsems[b % 2])
                        d_prev.wait()
                        pltpu.sync_copy(ubufs[(b - 1) % 2],
                                        acc.at[dbig.at[b - 1]], add=True)
                        d_prev = d_cur
                    d_prev.wait()
                    pltpu.sync_copy(ubufs[(GRP - 1) % 2],
                                    acc.at[dbig.at[GRP - 1]], add=True)
                    return carry

                lax.fori_loop(0, cnt_g, group, 0)

                @pl.when(s == 15)
                def _():
                    pltpu.sync_copy(src2.at[pl.ds(NGF * GRP, GRP), :], sbig)
                    pltpu.sync_copy(dst2.at[pl.ds(NGF * GRP, GRP), :], dbig)
                    for b in range(PARTIAL):
                        pltpu.async_copy(tab.at[sbig.at[b]], ubufs[0],
                                         gsems[0]).wait()
                        pltpu.sync_copy(ubufs[0], acc.at[dbig.at[b]],
                                        add=True)

                plsc.subcore_barrier()
                _writeback_rows(acc, wbuf, out, r0)

    return k


def _row_specs(width, n=1):
    return [pl.BlockSpec((BN, width), lambda i: (i, 0)) for _ in range(n)]


def _tc_pre1(deg_a, deg_b, x, w1):
    def body(da, db, xb, w, dinv_o, *houts):
        deg = da[:, 0:1] + db[:, 0:1] + 1.0
        dinv = lax.rsqrt(deg)
        h = jnp.dot(xb[...], w[...], preferred_element_type=F32,
                    precision=lax.Precision.HIGHEST) * dinv
        dinv_o[...] = dinv
        for i, o in enumerate(houts):
            o[...] = h[:, 16 * i:16 * (i + 1)]

    return pl.pallas_call(
        body,
        grid=(NB,),
        in_specs=(_row_specs(16) + _row_specs(16) + _row_specs(22)
                  + [pl.BlockSpec((22, 64), lambda i: (0, 0))]),
        out_specs=_row_specs(1) + _row_specs(16, 4),
        out_shape=([jax.ShapeDtypeStruct((NP, 1), F32)]
                   + [jax.ShapeDtypeStruct((NP, 16), F32)] * 4),
    )(deg_a, deg_b, x, w1)


def _tc_mid(S, hp, dinv, bvec, gvec, bevec, w, fbo):
    fout = 16 * fbo

    def body(*refs):
        s_refs = refs[0:4]
        h_refs = refs[4:8]
        dinv_r, b_r, g_r, be_r, w_r = refs[8:13]
        outs = refs[13:]
        sc = jnp.concatenate([r[...] for r in s_refs], axis=1)
        hc = jnp.concatenate([r[...] for r in h_refs], axis=1)
        dv = dinv_r[...]
        pre = dv * (sc + hc) + b_r[...]
        y = jnp.maximum(pre * (g_r[...] * (1.0 / np.sqrt(1.0 + EPS)))
                        + be_r[...], 0.0)
        hn = jnp.dot(y, w_r[...], preferred_element_type=F32,
                     precision=lax.Precision.HIGHEST) * dv
        for i, o in enumerate(outs):
            o[...] = hn[:, 16 * i:16 * (i + 1)]

    return pl.pallas_call(
        body,
        grid=(NB,),
        in_specs=(_row_specs(16, 4) + _row_specs(16, 4) + _row_specs(1)
                  + [pl.BlockSpec((1, 64), lambda i: (0, 0))] * 3
                  + [pl.BlockSpec((64, fout), lambda i: (0, 0))]),
        out_specs=_row_specs(16, fbo),
        out_shape=[jax.ShapeDtypeStruct((NP, 16), F32)] * fbo,
    )(*S, *hp, dinv, bvec, gvec, bevec, w)


def _tc_final(S, hp, dinv, b3, batch3):
    def body(*refs):
        s_refs = refs[0:8]
        h_refs = refs[8:16]
        dinv_r, b_r, batch_r, out_ref, acc, cntr = refs[16:]
        i = pl.program_id(0)

        @pl.when(i == 0)
        def _():
            acc[...] = jnp.zeros_like(acc)
            cntr[...] = jnp.zeros_like(cntr)

        sc = jnp.concatenate([r[...] for r in s_refs], axis=1)
        hc = jnp.concatenate([r[...] for r in h_refs], axis=1)
        pre = dinv_r[...] * (sc + hc) + b_r[...]
        y = jnp.maximum(pre, 0.0)
        oh = (lax.broadcasted_iota(jnp.int32, (G, BN), 0)
              == batch_r[0]).astype(F32)
        acc[...] += jnp.dot(oh, y, preferred_element_type=F32,
                            precision=lax.Precision.HIGHEST)
        cntr[...] += jnp.broadcast_to(
            jnp.sum(oh, axis=1, keepdims=True), (G, 128))
        out_ref[...] = acc[...] / jnp.maximum(cntr[...], 1.0)

    return pl.pallas_call(
        body,
        grid=(NB,),
        in_specs=(_row_specs(16, 8) + _row_specs(16, 8) + _row_specs(1)
                  + [pl.BlockSpec((1, 128), lambda i: (0, 0)),
                     pl.BlockSpec((1, 1, BN), lambda i: (i, 0, 0))]),
        out_specs=pl.BlockSpec((G, 128), lambda i: (0, 0)),
        out_shape=jax.ShapeDtypeStruct((G, 128), F32),
        scratch_shapes=[pltpu.VMEM((G, 128), F32), pltpu.VMEM((G, 128), F32)],
    )(*S, *hp, dinv, b3, batch3)


def kernel(x, edge_index, batch, W1, b1, g1, be1, W2, b2, g2, be2, W3, b3):
    src2 = jnp.pad(edge_index[0].reshape(NCHUNK, CH),
                   ((0, NCHUNK_PAD - NCHUNK), (0, 0)))
    dst2 = jnp.pad(edge_index[1].reshape(NCHUNK, CH),
                   ((0, NCHUNK_PAD - NCHUNK), (0, 0)))
    xp = jnp.pad(x, ((0, NP - N), (0, 0)))
    batchp = jnp.pad(batch, (0, NP - N), constant_values=-1)

    deg_a, deg_b = _make_sc_deg()(dst2)
    dinv, *hp1 = _tc_pre1(deg_a, deg_b, xp, W1)
    s1 = _make_sc_scatter(4)(src2, dst2, *hp1)
    hp2 = _tc_mid(s1, hp1, dinv, b1.reshape(1, 64), g1.reshape(1, 64),
                  be1.reshape(1, 64), W2, 4)
    s2 = _make_sc_scatter(4)(src2, dst2, *hp2)
    hp3 = _tc_mid(s2, hp2, dinv, b2.reshape(1, 64), g2.reshape(1, 64),
                  be2.reshape(1, 64), W3, 8)
    s3 = _make_sc_scatter(8)(src2, dst2, *hp3)
    return _tc_final(s3, hp3, dinv, b3.reshape(1, 128),
                     batch3=batchp.reshape(NB, 1, BN))
